# Initial kernel scaffold; baseline (speedup 1.0000x reference)
import jax
import jax.numpy as jnp
from jax.experimental import pallas as pl


def kernel(x, edge_index, edge_weight, W):
    raise NotImplementedError("write your pallas kernel here")


# speedup vs baseline: 4.2988x; 4.2988x over previous
import functools

import jax
import jax.numpy as jnp
from jax import lax
from jax.experimental import pallas as pl
from jax.experimental.pallas import tpu as pltpu
from jax.experimental.pallas import tpu_sc as plsc

NC = 2
NS = 16
NW = NC * NS
LANES = 16
CHUNK = 128


def _spmm_sc(x, src_r, dst_r, w_r, zeros_hbm, n_chunks, n_pad):
    _, d = x.shape
    rows_per_tile = n_pad // NS
    mesh = plsc.VectorSubcoreMesh(core_axis_name="c", subcore_axis_name="s")

    @functools.partial(
        pl.kernel,
        out_type=jax.ShapeDtypeStruct((NC, n_pad, d), jnp.float32),
        mesh=mesh,
        scratch_types=[
            pltpu.VMEM((n_chunks, CHUNK), jnp.int32),
            pltpu.VMEM((n_chunks, CHUNK), jnp.int32),
            pltpu.VMEM((n_chunks, CHUNK), jnp.float32),
            pltpu.VMEM((CHUNK, d), jnp.float32),
            pltpu.VMEM_SHARED((n_pad, d), jnp.float32),
            pltpu.SemaphoreType.DMA,
        ],
    )
    def spmm(x_hbm, src_hbm, dst_hbm, w_hbm, z_hbm, out_hbm,
             src_v, dst_v, w_v, rows_v, acc, sem):
        c = lax.axis_index("c")
        s = lax.axis_index("s")
        wid = s * NC + c
        base_r = s * rows_per_tile

        pltpu.sync_copy(src_hbm.at[wid], src_v)
        pltpu.sync_copy(dst_hbm.at[wid], dst_v)
        pltpu.sync_copy(w_hbm.at[wid], w_v)
        pltpu.sync_copy(z_hbm.at[pl.ds(base_r, rows_per_tile)],
                        acc.at[pl.ds(base_r, rows_per_tile)])
        plsc.subcore_barrier()

        def chunk_body(j, carry):
            pltpu.async_copy(x_hbm.at[src_v.at[j]], rows_v, sem).wait()

            def grp_body(g, carry2):
                w16 = w_v[j, pl.ds(g * LANES, LANES)]
                for r in range(LANES):
                    i = g * LANES + r
                    wv = w16[r]
                    for t in range(d // LANES):
                        sl = pl.ds(t * LANES, LANES)
                        rows_v[i, sl] = rows_v[i, sl] * wv
                return carry2

            lax.fori_loop(0, CHUNK // LANES, grp_body, 0, unroll=False)

            pltpu.sync_copy(rows_v, acc.at[dst_v.at[j]], add=True)
            return carry

        lax.fori_loop(0, n_chunks, chunk_body, 0, unroll=False)
        plsc.subcore_barrier()

        pltpu.sync_copy(acc.at[pl.ds(base_r, rows_per_tile)],
                        out_hbm.at[c, pl.ds(base_r, rows_per_tile)])

    return spmm(x, src_r, dst_r, w_r, zeros_hbm)


def _matmul_tc(partials, W):
    _, n, d = partials.shape
    bn = 512
    assert n % bn == 0

    def body(p_ref, w_ref, o_ref):
        ps = p_ref[0] + p_ref[1]
        o_ref[...] = jnp.dot(ps, w_ref[...], preferred_element_type=jnp.float32)

    return pl.pallas_call(
        body,
        grid=(n // bn,),
        in_specs=[
            pl.BlockSpec((NC, bn, d), lambda i: (0, i, 0)),
            pl.BlockSpec((d, d), lambda i: (0, 0)),
        ],
        out_specs=pl.BlockSpec((bn, d), lambda i: (i, 0)),
        out_shape=jax.ShapeDtypeStruct((n, d), jnp.float32),
    )(partials, W)


def kernel(x, edge_index, edge_weight, W):
    n, d = x.shape
    e = edge_weight.shape[0]
    n_pad = -(-n // 1024) * 1024

    n_chunks = -(-e // (NW * CHUNK))
    e_pad = NW * n_chunks * CHUNK
    src = edge_index[0]
    dst = edge_index[1]
    src_r = jnp.zeros((e_pad,), jnp.int32).at[:e].set(src).reshape(NW, n_chunks, CHUNK)
    dst_r = jnp.zeros((e_pad,), jnp.int32).at[:e].set(dst).reshape(NW, n_chunks, CHUNK)
    w_r = jnp.zeros((e_pad,), jnp.float32).at[:e].set(edge_weight).reshape(NW, n_chunks, CHUNK)
    zeros_hbm = jnp.zeros((n_pad, d), jnp.float32)

    partials = _spmm_sc(x, src_r, dst_r, w_r, zeros_hbm, n_chunks, n_pad)
    return _matmul_tc(partials, W)[:n]
